# baseline (speedup 1.0000x reference)
import jax
import jax.numpy as jnp
from jax.experimental import pallas as pl

N = 10000
G = 64
H1 = 64
H2 = 16
D_OUT = 10


def _tconv(x, edge_index, edge_attr, Wq, bq, Wk, bk, Wv, bv, We, Ws, bs, out_ch):
    src = edge_index[0]
    dst = edge_index[1]
    n = x.shape[0]
    q = x @ Wq + bq
    k = x @ Wk + bk
    v = x @ Wv + bv
    e = edge_attr @ We
    k_j = k[src] + e
    v_j = v[src] + e
    alpha = jnp.sum(q[dst] * k_j, axis=-1) / jnp.sqrt(float(out_ch))
    amax = jax.ops.segment_max(alpha, dst, num_segments=n)
    amax = jnp.where(jnp.isfinite(amax), amax, 0.0)
    ex = jnp.exp(alpha - amax[dst])
    denom = jax.ops.segment_sum(ex, dst, num_segments=n)
    w = ex / (denom[dst] + 1e-16)
    out = jax.ops.segment_sum(v_j * w[:, None], dst, num_segments=n)
    return out + (x @ Ws + bs)


def _ln(h, g, b):
    mu = jnp.mean(h, axis=-1, keepdims=True)
    var = jnp.var(h, axis=-1, keepdims=True)
    return (h - mu) / jnp.sqrt(var + 1e-5) * g + b


def _final_linear_kernel(z_ref, w_ref, b_ref, o_ref):
    o_ref[...] = z_ref[...] @ w_ref[...] + b_ref[...]


def kernel(x, edge_index, edge_attr, batch, Wq1, bq1, Wk1, bk1, Wv1, bv1, We1, Ws1, bs1, g1, be1, Wq2, bq2, Wk2, bk2, Wv2, bv2, We2, Ws2, bs2, g2, be2, Wf, bf):
    h = _tconv(x, edge_index, edge_attr, Wq1, bq1, Wk1, bk1, Wv1, bv1, We1, Ws1, bs1, H1)
    h = jax.nn.relu(_ln(h, g1, be1))
    h = _tconv(h, edge_index, edge_attr, Wq2, bq2, Wk2, bk2, Wv2, bv2, We2, Ws2, bs2, H2)
    h = jax.nn.relu(_ln(h, g2, be2))
    sums = jax.ops.segment_sum(h, batch, num_segments=G)
    cnt = jax.ops.segment_sum(jnp.ones((h.shape[0],), jnp.float32), batch, num_segments=G)
    z = sums / jnp.maximum(cnt, 1.0)[:, None]
    return pl.pallas_call(
        _final_linear_kernel,
        out_shape=jax.ShapeDtypeStruct((G, D_OUT), jnp.float32),
    )(z, Wf, bf[None, :])


# speedup vs baseline: 7.1454x; 7.1454x over previous
import functools

import jax
import jax.numpy as jnp
from jax.experimental import pallas as pl
from jax.experimental.pallas import tpu as pltpu
from jax.experimental.pallas import tpu_sc as plsc

N = 10000
E = 320000
G = 64
D_IN = 128
D_EDGE = 16
H1 = 64
H2 = 16
D_OUT = 10

NC = 2
NS = 16
NW = NC * NS
C = 80
NCH = E // NW // C
NCHB = E // NS // C

DA = 128
HALF = 5120
NPH = 5248
RPSH = NPH // NS
TRASH = 5240

_EPS_DEN = 1e-16
_EPS_LN = 1e-5

_GDN = jax.lax.GatherDimensionNumbers(
    offset_dims=(), collapsed_slice_dims=(0,), start_index_map=(0,))


def _lane_shuffle(v, idx):
    return jax.lax.gather(v, idx[:, None], _GDN, (1,),
                          mode=jax.lax.GatherScatterMode.PROMISE_IN_BOUNDS)


def _lane_sum(v):
    for sh in (8, 4, 2, 1):
        v = v + _lane_shuffle(v, jax.lax.iota(jnp.int32, 16) ^ sh)
    return v


def _make_edgeA(H, scale):
    mesh = plsc.VectorSubcoreMesh(core_axis_name="c", subcore_axis_name="s")
    nh = H // 16

    @functools.partial(
        pl.kernel,
        mesh=mesh,
        out_type=jax.ShapeDtypeStruct((NW, NCH, C, DA), jnp.float32),
        scratch_types=[
            pltpu.VMEM((NCH, C), jnp.int32),
            pltpu.VMEM((NCH, C), jnp.int32),
            pltpu.VMEM((C, DA), jnp.float32),
            pltpu.VMEM((C, DA), jnp.float32),
            pltpu.VMEM((C, D_EDGE), jnp.float32),
            pltpu.VMEM((C, DA), jnp.float32),
            pltpu.VMEM((16,), jnp.float32),
            pltpu.SemaphoreType.DMA,
            pltpu.SemaphoreType.DMA,
        ],
    )
    def edgeA(dnode, snode, ea4, src3, dst3, bvec_in, out,
              srcv, dstv, drows, srows, eav, contrib, bvec, sem1, sem2):
        cid = jax.lax.axis_index("c")
        sid = jax.lax.axis_index("s")
        wid = sid * NC + cid

        pltpu.sync_copy(src3.at[wid], srcv)
        pltpu.sync_copy(dst3.at[wid], dstv)
        pltpu.sync_copy(bvec_in, bvec)
        b_s = bvec[...]
        lane0 = jnp.where(jax.lax.iota(jnp.int32, 16) == 0, 1.0, 0.0)
        zero16 = jnp.zeros((16,), jnp.float32)

        def zpad(r, carry):
            for c2 in range((DA - H - 32) // 16):
                contrib[r, pl.ds(H + 32 + c2 * 16, 16)] = zero16
            return carry
        jax.lax.fori_loop(0, C, zpad, 0)

        def edge(r, carry):
            al = drows[r, pl.ds(H, 16)] * eav[r, :]
            for c2 in range(nh):
                al = al + (drows[r, pl.ds(c2 * 16, 16)]
                           * srows[r, pl.ds(c2 * 16, 16)])
            ex = jnp.exp(_lane_sum(al) * scale - b_s)
            for c2 in range(nh):
                contrib[r, pl.ds(c2 * 16, 16)] = (
                    ex * srows[r, pl.ds(H + c2 * 16, 16)])
            contrib[r, pl.ds(H, 16)] = ex * eav[r, :]
            contrib[r, pl.ds(H + 16, 16)] = ex * lane0
            return carry

        def chunk(j, carry):
            pltpu.async_copy(dnode.at[dstv.at[j]], drows, sem1).wait()
            pltpu.async_copy(snode.at[srcv.at[j]], srows, sem2).wait()
            pltpu.sync_copy(ea4.at[wid, j], eav)
            jax.lax.fori_loop(0, C, edge, 0)
            pltpu.sync_copy(contrib, out.at[wid, j])
            return carry
        jax.lax.fori_loop(0, NCH, chunk, 0)

    return edgeA


def _make_edgeB():
    mesh = plsc.VectorSubcoreMesh(core_axis_name="c", subcore_axis_name="s")

    @functools.partial(
        pl.kernel,
        mesh=mesh,
        out_type=jax.ShapeDtypeStruct((NC, NPH, DA), jnp.float32),
        scratch_types=[
            pltpu.VMEM((NCHB, C), jnp.int32),
            pltpu.VMEM((C, DA), jnp.float32),
            pltpu.VMEM((C,), jnp.int32),
            pltpu.VMEM_SHARED((NPH, DA), jnp.float32),
            pltpu.SemaphoreType.DMA,
        ],
    )
    def edgeB(cb4, dstb, zrows, out, dstv, rows, idxv, acc, sem):
        cid = jax.lax.axis_index("c")
        sid = jax.lax.axis_index("s")
        base = cid * HALF

        pltpu.sync_copy(dstb.at[sid], dstv)
        pltpu.sync_copy(zrows, acc.at[pl.ds(sid * RPSH, RPSH)])
        plsc.subcore_barrier()

        def chunk(j, carry):
            pltpu.async_copy(cb4.at[sid, j], rows, sem).wait()
            for g in range(C // 16):
                v = dstv[j, pl.ds(g * 16, 16)] - base
                ok = jnp.logical_and(v >= 0, v < HALF)
                idxv[pl.ds(g * 16, 16)] = jnp.where(ok, v, TRASH)
            pltpu.sync_copy(rows, acc.at[idxv], add=True)
            return carry
        jax.lax.fori_loop(0, NCHB, chunk, 0)

        plsc.subcore_barrier()
        pltpu.sync_copy(acc.at[pl.ds(sid * RPSH, RPSH)],
                        out.at[cid, pl.ds(sid * RPSH, RPSH)])

    return edgeB


_edgeA1 = _make_edgeA(H1, 1.0 / 8.0)
_edgeA2 = _make_edgeA(H2, 1.0 / 4.0)
_edgeB = _make_edgeB()


def _eamax_body(ea_ref, out_ref, mx_ref):
    i = pl.program_id(0)
    ea = ea_ref[...]
    m = jnp.max(jnp.sum(ea * ea, axis=1))
    prev = jnp.where(i == 0, -jnp.inf, mx_ref[0])
    mx_ref[0] = jnp.maximum(prev, m)
    @pl.when(i == pl.num_programs(0) - 1)
    def _():
        out_ref[...] = jnp.full((1, 128), mx_ref[0], jnp.float32)


def _eamax(edge_attr):
    return pl.pallas_call(
        _eamax_body,
        grid=(16,),
        in_specs=[pl.BlockSpec((E // 16, D_EDGE), lambda i: (i, 0))],
        out_specs=pl.BlockSpec((1, 128), lambda i: (0, 0)),
        out_shape=jax.ShapeDtypeStruct((1, 128), jnp.float32),
        scratch_shapes=[pltpu.SMEM((4,), jnp.float32)],
    )(edge_attr)


def _prep1_body(x_ref, wq, bq, wk, bk, wv, bv, wet, ws, bs, mea_ref,
                dnode_ref, snode_ref, skip_ref, bvec_ref, mx_ref):
    i = pl.program_id(0)
    xb = x_ref[...]
    q = xb @ wq[...] + bq[...]
    k = xb @ wk[...] + bk[...]
    v = xb @ wv[...] + bv[...]
    qe = q @ wet[...]
    pad = jnp.zeros((q.shape[0], DA - H1 - D_EDGE), jnp.float32)
    dnode_ref[...] = jnp.concatenate([q, qe, pad], axis=1)
    snode_ref[...] = jnp.concatenate([k, v], axis=1)
    skip_ref[...] = xb @ ws[...] + bs[...]
    mq = jnp.max(jnp.sum(q * q, axis=1))
    mk = jnp.max(jnp.sum(k * k, axis=1))
    mqe = jnp.max(jnp.sum(qe * qe, axis=1))
    first = i == 0
    mx_ref[0] = jnp.maximum(jnp.where(first, -jnp.inf, mx_ref[0]), mq)
    mx_ref[1] = jnp.maximum(jnp.where(first, -jnp.inf, mx_ref[1]), mk)
    mx_ref[2] = jnp.maximum(jnp.where(first, -jnp.inf, mx_ref[2]), mqe)
    @pl.when(i == pl.num_programs(0) - 1)
    def _():
        mea = mea_ref[...]
        b = (jnp.sqrt(mx_ref[0] * mx_ref[1]) + jnp.sqrt(mx_ref[2] * mea)) / 8.0
        bvec_ref[...] = b


def _prep1(x, Wq1, bq1, Wk1, bk1, Wv1, bv1, We1T, Ws1, bs1, mea):
    nb = 10
    blk = N // nb
    full = lambda r, c: pl.BlockSpec((r, c), lambda i: (0, 0))
    return pl.pallas_call(
        _prep1_body,
        grid=(nb,),
        in_specs=[
            pl.BlockSpec((blk, D_IN), lambda i: (i, 0)),
            full(D_IN, H1), full(1, H1),
            full(D_IN, H1), full(1, H1),
            full(D_IN, H1), full(1, H1),
            full(H1, D_EDGE),
            full(D_IN, H1), full(1, H1),
            full(1, 128),
        ],
        out_specs=[
            pl.BlockSpec((blk, DA), lambda i: (i, 0)),
            pl.BlockSpec((blk, DA), lambda i: (i, 0)),
            pl.BlockSpec((blk, H1), lambda i: (i, 0)),
            pl.BlockSpec((1, 128), lambda i: (0, 0)),
        ],
        out_shape=[
            jax.ShapeDtypeStruct((N, DA), jnp.float32),
            jax.ShapeDtypeStruct((N, DA), jnp.float32),
            jax.ShapeDtypeStruct((N, H1), jnp.float32),
            jax.ShapeDtypeStruct((1, 128), jnp.float32),
        ],
        scratch_shapes=[pltpu.SMEM((4,), jnp.float32)],
    )(x, Wq1, bq1, Wk1, bk1, Wv1, bv1, We1T, Ws1, bs1, mea)


def _combine1_body(a_ref, we1, skip_ref, g1, be1,
                   wq, bq, wk, bk, wv, bv, wet, ws, bs, mea_ref,
                   dnode_ref, snode_ref, skip2_ref, bvec_ref, mx_ref):
    i = pl.program_id(0)
    a = a_ref[...]
    sv = a[:, :H1]
    sa = a[:, H1:H1 + D_EDGE]
    den = a[:, H1 + D_EDGE:H1 + D_EDGE + 1]
    o = (sv + sa @ we1[...]) / (den + _EPS_DEN) + skip_ref[...]
    mu = jnp.mean(o, axis=1, keepdims=True)
    var = jnp.mean((o - mu) * (o - mu), axis=1, keepdims=True)
    h = (o - mu) / jnp.sqrt(var + _EPS_LN) * g1[...] + be1[...]
    h = jnp.maximum(h, 0.0)
    q = h @ wq[...] + bq[...]
    k = h @ wk[...] + bk[...]
    v = h @ wv[...] + bv[...]
    qe = q @ wet[...]
    padd = jnp.zeros((q.shape[0], DA - H2 - D_EDGE), jnp.float32)
    pads = jnp.zeros((q.shape[0], DA - 2 * H2), jnp.float32)
    dnode_ref[...] = jnp.concatenate([q, qe, padd], axis=1)
    snode_ref[...] = jnp.concatenate([k, v, pads], axis=1)
    skip2_ref[...] = h @ ws[...] + bs[...]
    mq = jnp.max(jnp.sum(q * q, axis=1))
    mk = jnp.max(jnp.sum(k * k, axis=1))
    mqe = jnp.max(jnp.sum(qe * qe, axis=1))
    first = i == 0
    mx_ref[0] = jnp.maximum(jnp.where(first, -jnp.inf, mx_ref[0]), mq)
    mx_ref[1] = jnp.maximum(jnp.where(first, -jnp.inf, mx_ref[1]), mk)
    mx_ref[2] = jnp.maximum(jnp.where(first, -jnp.inf, mx_ref[2]), mqe)
    @pl.when(i == pl.num_programs(0) - 1)
    def _():
        mea = mea_ref[...]
        b = (jnp.sqrt(mx_ref[0] * mx_ref[1]) + jnp.sqrt(mx_ref[2] * mea)) / 4.0
        bvec_ref[...] = b


def _combine1(acc, We1, skip1, g1, be1,
              Wq2, bq2, Wk2, bk2, Wv2, bv2, We2T, Ws2, bs2, mea):
    nb = 10
    blk = N // nb
    full = lambda r, c: pl.BlockSpec((r, c), lambda i: (0, 0))
    return pl.pallas_call(
        _combine1_body,
        grid=(nb,),
        in_specs=[
            pl.BlockSpec((blk, DA), lambda i: (i, 0)),
            full(D_EDGE, H1),
            pl.BlockSpec((blk, H1), lambda i: (i, 0)),
            full(1, H1), full(1, H1),
            full(H1, H2), full(1, H2),
            full(H1, H2), full(1, H2),
            full(H1, H2), full(1, H2),
            full(H2, D_EDGE),
            full(H1, H2), full(1, H2),
            full(1, 128),
        ],
        out_specs=[
            pl.BlockSpec((blk, DA), lambda i: (i, 0)),
            pl.BlockSpec((blk, DA), lambda i: (i, 0)),
            pl.BlockSpec((blk, H2), lambda i: (i, 0)),
            pl.BlockSpec((1, 128), lambda i: (0, 0)),
        ],
        out_shape=[
            jax.ShapeDtypeStruct((N, DA), jnp.float32),
            jax.ShapeDtypeStruct((N, DA), jnp.float32),
            jax.ShapeDtypeStruct((N, H2), jnp.float32),
            jax.ShapeDtypeStruct((1, 128), jnp.float32),
        ],
        scratch_shapes=[pltpu.SMEM((4,), jnp.float32)],
    )(acc, We1, skip1, g1, be1,
      Wq2, bq2, Wk2, bk2, Wv2, bv2, We2T, Ws2, bs2, mea)


def _final_body(a_ref, we2, skip_ref, g2, be2, batch_ref, wf, bf,
                out_ref):
    a = a_ref[...]
    sv = a[:, :H2]
    sa = a[:, H2:H2 + D_EDGE]
    den = a[:, H2 + D_EDGE:H2 + D_EDGE + 1]
    o = (sv + sa @ we2[...]) / (den + _EPS_DEN) + skip_ref[...]
    mu = jnp.mean(o, axis=1, keepdims=True)
    var = jnp.mean((o - mu) * (o - mu), axis=1, keepdims=True)
    h = (o - mu) / jnp.sqrt(var + _EPS_LN) * g2[...] + be2[...]
    h = jnp.maximum(h, 0.0)
    gids = jax.lax.broadcasted_iota(jnp.int32, (N, G), 1)
    onehot = (batch_ref[...] == gids).astype(jnp.float32)
    sums = jax.lax.dot_general(onehot, h, (((0,), (0,)), ((), ())))
    cnt = jnp.sum(onehot, axis=0)
    z = sums / jnp.maximum(cnt, 1.0)[:, None]
    out_ref[...] = z @ wf[...] + bf[...]


def _final(acc, We2, skip2, g2, be2, batch2d, Wf, bf):
    return pl.pallas_call(
        _final_body,
        out_shape=jax.ShapeDtypeStruct((G, D_OUT), jnp.float32),
    )(acc, We2, skip2, g2, be2, batch2d, Wf, bf)


def kernel(x, edge_index, edge_attr, batch,
           Wq1, bq1, Wk1, bk1, Wv1, bv1, We1, Ws1, bs1, g1, be1,
           Wq2, bq2, Wk2, bk2, Wv2, bv2, We2, Ws2, bs2, g2, be2, Wf, bf):
    src3 = edge_index[0].reshape(NW, NCH, C)
    dst3 = edge_index[1].reshape(NW, NCH, C)
    dstb = edge_index[1].reshape(NS, NCHB, C)
    ea4 = edge_attr.reshape(NW, NCH, C, D_EDGE)
    zrows = jnp.zeros((RPSH, DA), jnp.float32)

    mea = _eamax(edge_attr)
    dnode1, snode1, skip1, bvec1 = _prep1(
        x, Wq1, bq1[None, :], Wk1, bk1[None, :], Wv1, bv1[None, :],
        We1.T, Ws1, bs1[None, :], mea)

    cb1 = _edgeA1(dnode1, snode1, ea4, src3, dst3, bvec1[0, :16])
    accB1 = _edgeB(cb1.reshape(NS, NCHB, C, DA), dstb, zrows)
    acc1 = jnp.concatenate([accB1[0, :HALF], accB1[1, :N - HALF]], axis=0)

    dnode2, snode2, skip2, bvec2 = _combine1(
        acc1, We1, skip1, g1[None, :], be1[None, :],
        Wq2, bq2[None, :], Wk2, bk2[None, :], Wv2, bv2[None, :],
        We2.T, Ws2, bs2[None, :], mea)

    cb2 = _edgeA2(dnode2, snode2, ea4, src3, dst3, bvec2[0, :16])
    accB2 = _edgeB(cb2.reshape(NS, NCHB, C, DA), dstb, zrows)
    acc2 = jnp.concatenate([accB2[0, :HALF], accB2[1, :N - HALF]], axis=0)

    return _final(acc2, We2, skip2,
                  g2[None, :], be2[None, :], batch.reshape(N, 1), Wf, bf[None, :])
